# initial kernel scaffold (unmeasured)
import jax
import jax.numpy as jnp
from jax import lax
from jax.experimental import pallas as pl
from jax.experimental.pallas import tpu as pltpu

SCALE = 128.0 ** -0.5


def _partial_body(q_ref, k_ref, v_ref, onum_ref, l_ref):
    hi = pl.program_id(1)
    q = q_ref[0, :, 0, :].astype(jnp.bfloat16)
    k = k_ref[0, :, 0, :].astype(jnp.bfloat16)
    v = v_ref[0, :, 0, :].astype(jnp.bfloat16)
    s = lax.dot_general(
        q, k, (((1,), (1,)), ((), ())), preferred_element_type=jnp.float32
    )
    p = jnp.exp(s * SCALE)
    onum = lax.dot_general(
        p.astype(jnp.bfloat16), v, (((1,), (0,)), ((), ())),
        preferred_element_type=jnp.float32,
    )
    onum_ref[0, :, 0, :] = onum
    ones = jnp.ones((1, p.shape[1]), dtype=jnp.float32)
    lrow = lax.dot_general(
        ones, p, (((1,), (1,)), ((), ())), preferred_element_type=jnp.float32
    )
    l_ref[0, pl.ds(hi, 1), :] = lrow


def _combine_body(
    onum_ref, l_ref, osum_ref, lsum_ref, obuf, lbuf, send_sems, recv_sems
):
    my_x = lax.axis_index("x")
    my_y = lax.axis_index("y")
    my_z = lax.axis_index("z")
    peer_a = my_z + 1 - 2 * lax.rem(my_z, 2)
    peer_b = lax.rem(my_z + 2, 4)

    barrier = pltpu.get_barrier_semaphore()
    for pz in (peer_a, peer_b):
        pl.semaphore_signal(
            barrier, inc=1, device_id=(my_x, my_y, pz),
            device_id_type=pl.DeviceIdType.MESH,
        )
    pl.semaphore_wait(barrier, 2)

    osum_ref[...] = onum_ref[...]
    lsum_ref[...] = l_ref[...]

    for step, pz in enumerate((peer_a, peer_b)):
        target = (my_x, my_y, pz)
        o_rdma = pltpu.make_async_remote_copy(
            src_ref=osum_ref, dst_ref=obuf.at[step],
            send_sem=send_sems.at[2 * step], recv_sem=recv_sems.at[2 * step],
            device_id=target, device_id_type=pl.DeviceIdType.MESH,
        )
        l_rdma = pltpu.make_async_remote_copy(
            src_ref=lsum_ref, dst_ref=lbuf.at[step],
            send_sem=send_sems.at[2 * step + 1],
            recv_sem=recv_sems.at[2 * step + 1],
            device_id=target, device_id_type=pl.DeviceIdType.MESH,
        )
        o_rdma.start()
        l_rdma.start()
        o_rdma.wait()
        l_rdma.wait()
        osum_ref[...] = osum_ref[...] + obuf[step]
        lsum_ref[...] = lsum_ref[...] + lbuf[step]


def kernel(Q, K, V):
    B, SQ, H, D = Q.shape
    KV = K.shape[1]

    onum, l = pl.pallas_call(
        _partial_body,
        grid=(B, H),
        in_specs=[
            pl.BlockSpec((1, SQ, 1, D), lambda bi, hi: (bi, 0, hi, 0)),
            pl.BlockSpec((1, KV, 1, D), lambda bi, hi: (bi, 0, hi, 0)),
            pl.BlockSpec((1, KV, 1, D), lambda bi, hi: (bi, 0, hi, 0)),
        ],
        out_specs=[
            pl.BlockSpec((1, SQ, 1, D), lambda bi, hi: (bi, 0, hi, 0)),
            pl.BlockSpec((1, H, SQ), lambda bi, hi: (bi, 0, 0)),
        ],
        out_shape=[
            jax.ShapeDtypeStruct((B, SQ, H, D), jnp.float32),
            jax.ShapeDtypeStruct((B, H, SQ), jnp.float32),
        ],
    )(Q, K, V)

    osum, lsum = pl.pallas_call(
        _combine_body,
        in_specs=[
            pl.BlockSpec(memory_space=pltpu.VMEM),
            pl.BlockSpec(memory_space=pltpu.VMEM),
        ],
        out_specs=[
            pl.BlockSpec(memory_space=pltpu.VMEM),
            pl.BlockSpec(memory_space=pltpu.VMEM),
        ],
        out_shape=[
            jax.ShapeDtypeStruct((B, SQ, H, D), jnp.float32),
            jax.ShapeDtypeStruct((B, H, SQ), jnp.float32),
        ],
        scratch_shapes=[
            pltpu.VMEM((2, B, SQ, H, D), jnp.float32),
            pltpu.VMEM((2, B, H, SQ), jnp.float32),
            pltpu.SemaphoreType.DMA((4,)),
            pltpu.SemaphoreType.DMA((4,)),
        ],
        compiler_params=pltpu.CompilerParams(collective_id=0),
    )(onum, l)

    return osum / jnp.transpose(lsum, (0, 2, 1))[..., None]


# baseline (device time: 168611 ns/iter reference)
import jax
import jax.numpy as jnp
from jax import lax
from jax.experimental import pallas as pl
from jax.experimental.pallas import tpu as pltpu

SCALE = 128.0 ** -0.5


def _partial_body(q_ref, k_ref, v_ref, onum_ref, l_ref):
    n_heads = q_ref.shape[2]
    for h in range(n_heads):
        q = q_ref[0, :, h, :].astype(jnp.bfloat16)
        k = k_ref[0, :, h, :].astype(jnp.bfloat16)
        v = v_ref[0, :, h, :].astype(jnp.bfloat16)
        s = lax.dot_general(
            q, k, (((1,), (1,)), ((), ())), preferred_element_type=jnp.float32
        )
        p = jnp.exp(s * SCALE)
        onum = lax.dot_general(
            p.astype(jnp.bfloat16), v, (((1,), (0,)), ((), ())),
            preferred_element_type=jnp.float32,
        )
        onum_ref[0, :, h, :] = onum
        ones = jnp.ones((1, p.shape[1]), dtype=jnp.float32)
        lrow = lax.dot_general(
            ones, p, (((1,), (1,)), ((), ())),
            preferred_element_type=jnp.float32,
        )
        l_ref[0, pl.ds(h, 1), :] = lrow


def _combine_body(
    onum_ref, l_ref, osum_ref, lsum_ref, obuf, lbuf, send_sems, recv_sems
):
    my_x = lax.axis_index("x")
    my_y = lax.axis_index("y")
    my_z = lax.axis_index("z")
    peer_a = my_z + 1 - 2 * lax.rem(my_z, 2)
    peer_b = lax.rem(my_z + 2, 4)

    barrier = pltpu.get_barrier_semaphore()
    for pz in (peer_a, peer_b):
        pl.semaphore_signal(
            barrier, inc=1, device_id=(my_x, my_y, pz),
            device_id_type=pl.DeviceIdType.MESH,
        )
    pl.semaphore_wait(barrier, 2)

    osum_ref[...] = onum_ref[...]
    lsum_ref[...] = l_ref[...]

    for step, pz in enumerate((peer_a, peer_b)):
        target = (my_x, my_y, pz)
        o_rdma = pltpu.make_async_remote_copy(
            src_ref=osum_ref, dst_ref=obuf.at[step],
            send_sem=send_sems.at[2 * step], recv_sem=recv_sems.at[2 * step],
            device_id=target, device_id_type=pl.DeviceIdType.MESH,
        )
        l_rdma = pltpu.make_async_remote_copy(
            src_ref=lsum_ref, dst_ref=lbuf.at[step],
            send_sem=send_sems.at[2 * step + 1],
            recv_sem=recv_sems.at[2 * step + 1],
            device_id=target, device_id_type=pl.DeviceIdType.MESH,
        )
        o_rdma.start()
        l_rdma.start()
        o_rdma.wait()
        l_rdma.wait()
        osum_ref[...] = osum_ref[...] + obuf[step]
        lsum_ref[...] = lsum_ref[...] + lbuf[step]


def kernel(Q, K, V):
    B, SQ, H, D = Q.shape
    KV = K.shape[1]

    onum, l = pl.pallas_call(
        _partial_body,
        grid=(B,),
        in_specs=[
            pl.BlockSpec((1, SQ, H, D), lambda bi: (bi, 0, 0, 0)),
            pl.BlockSpec((1, KV, H, D), lambda bi: (bi, 0, 0, 0)),
            pl.BlockSpec((1, KV, H, D), lambda bi: (bi, 0, 0, 0)),
        ],
        out_specs=[
            pl.BlockSpec((1, SQ, H, D), lambda bi: (bi, 0, 0, 0)),
            pl.BlockSpec((1, H, SQ), lambda bi: (bi, 0, 0)),
        ],
        out_shape=[
            jax.ShapeDtypeStruct((B, SQ, H, D), jnp.float32),
            jax.ShapeDtypeStruct((B, H, SQ), jnp.float32),
        ],
    )(Q, K, V)

    osum, lsum = pl.pallas_call(
        _combine_body,
        in_specs=[
            pl.BlockSpec(memory_space=pltpu.VMEM),
            pl.BlockSpec(memory_space=pltpu.VMEM),
        ],
        out_specs=[
            pl.BlockSpec(memory_space=pltpu.VMEM),
            pl.BlockSpec(memory_space=pltpu.VMEM),
        ],
        out_shape=[
            jax.ShapeDtypeStruct((B, SQ, H, D), jnp.float32),
            jax.ShapeDtypeStruct((B, H, SQ), jnp.float32),
        ],
        scratch_shapes=[
            pltpu.VMEM((2, B, SQ, H, D), jnp.float32),
            pltpu.VMEM((2, B, H, SQ), jnp.float32),
            pltpu.SemaphoreType.DMA((4,)),
            pltpu.SemaphoreType.DMA((4,)),
        ],
        compiler_params=pltpu.CompilerParams(collective_id=0),
    )(onum, l)

    return osum / jnp.transpose(lsum, (0, 2, 1))[..., None]


# device time: 126822 ns/iter; 1.3295x vs baseline; 1.3295x over previous
import jax
import jax.numpy as jnp
from jax import lax
from jax.experimental import pallas as pl
from jax.experimental.pallas import tpu as pltpu

SCALE = 128.0 ** -0.5


def _partial_body(q_ref, k_ref, v_ref, onum_ref, l_ref):
    q = q_ref[0, 0]
    k = k_ref[0, 0]
    v = v_ref[0, 0]
    s = lax.dot_general(
        q, k, (((1,), (1,)), ((), ())), preferred_element_type=jnp.float32
    )
    p = jnp.exp(s * SCALE)
    onum = lax.dot_general(
        p.astype(jnp.bfloat16), v, (((1,), (0,)), ((), ())),
        preferred_element_type=jnp.float32,
    )
    onum_ref[0, 0] = onum
    l_ref[0, 0] = jnp.sum(p, axis=1, keepdims=True)


def _combine_body(onum_ref, l_ref, out_ref, lacc, obuf, lbuf, send_sems, recv_sems):
    my_x = lax.axis_index("x")
    my_y = lax.axis_index("y")
    my_z = lax.axis_index("z")
    partners = (
        (1 - my_x, my_y, my_z),
        (my_x, 1 - my_y, my_z),
        (my_x, my_y, my_z + 1 - 2 * lax.rem(my_z, 2)),
        (my_x, my_y, lax.rem(my_z + 2, 4)),
    )

    barrier = pltpu.get_barrier_semaphore()
    for tgt in partners:
        pl.semaphore_signal(
            barrier, inc=1, device_id=tgt, device_id_type=pl.DeviceIdType.MESH
        )
    pl.semaphore_wait(barrier, len(partners))

    out_ref[...] = onum_ref[...]
    lacc[...] = l_ref[...]

    for step, tgt in enumerate(partners):
        o_rdma = pltpu.make_async_remote_copy(
            src_ref=out_ref, dst_ref=obuf.at[step],
            send_sem=send_sems.at[2 * step], recv_sem=recv_sems.at[2 * step],
            device_id=tgt, device_id_type=pl.DeviceIdType.MESH,
        )
        l_rdma = pltpu.make_async_remote_copy(
            src_ref=lacc, dst_ref=lbuf.at[step],
            send_sem=send_sems.at[2 * step + 1],
            recv_sem=recv_sems.at[2 * step + 1],
            device_id=tgt, device_id_type=pl.DeviceIdType.MESH,
        )
        o_rdma.start()
        l_rdma.start()
        o_rdma.wait()
        l_rdma.wait()
        out_ref[...] = out_ref[...] + obuf[step]
        lacc[...] = lacc[...] + lbuf[step]

    out_ref[...] = out_ref[...] / lacc[...]


def kernel(Q, K, V):
    B, SQ, H, D = Q.shape
    KV = K.shape[1]
    KVQ = KV // 4

    rank = 2 * lax.axis_index("x") + lax.axis_index("y")
    Kq = lax.dynamic_slice(K, (0, rank * KVQ, 0, 0), (B, KVQ, H, D))
    Vq = lax.dynamic_slice(V, (0, rank * KVQ, 0, 0), (B, KVQ, H, D))
    Kt = jnp.transpose(Kq, (0, 2, 1, 3)).astype(jnp.bfloat16)
    Vt = jnp.transpose(Vq, (0, 2, 1, 3)).astype(jnp.bfloat16)
    Qt = jnp.transpose(Q, (0, 2, 1, 3)).astype(jnp.bfloat16)

    onum, l = pl.pallas_call(
        _partial_body,
        grid=(B, H),
        in_specs=[
            pl.BlockSpec((1, 1, SQ, D), lambda bi, hi: (bi, hi, 0, 0)),
            pl.BlockSpec((1, 1, KVQ, D), lambda bi, hi: (bi, hi, 0, 0)),
            pl.BlockSpec((1, 1, KVQ, D), lambda bi, hi: (bi, hi, 0, 0)),
        ],
        out_specs=[
            pl.BlockSpec((1, 1, SQ, D), lambda bi, hi: (bi, hi, 0, 0)),
            pl.BlockSpec((1, 1, SQ, 1), lambda bi, hi: (bi, hi, 0, 0)),
        ],
        out_shape=[
            jax.ShapeDtypeStruct((B, H, SQ, D), jnp.float32),
            jax.ShapeDtypeStruct((B, H, SQ, 1), jnp.float32),
        ],
    )(Qt, Kt, Vt)

    out = pl.pallas_call(
        _combine_body,
        in_specs=[
            pl.BlockSpec(memory_space=pltpu.VMEM),
            pl.BlockSpec(memory_space=pltpu.VMEM),
        ],
        out_specs=pl.BlockSpec(memory_space=pltpu.VMEM),
        out_shape=jax.ShapeDtypeStruct((B, H, SQ, D), jnp.float32),
        scratch_shapes=[
            pltpu.VMEM((B, H, SQ, 1), jnp.float32),
            pltpu.VMEM((4, B, H, SQ, D), jnp.float32),
            pltpu.VMEM((4, B, H, SQ, 1), jnp.float32),
            pltpu.SemaphoreType.DMA((8,)),
            pltpu.SemaphoreType.DMA((8,)),
        ],
        compiler_params=pltpu.CompilerParams(collective_id=0),
    )(onum, l)

    return jnp.transpose(out, (0, 2, 1, 3))


# device time: 71264 ns/iter; 2.3660x vs baseline; 1.7796x over previous
import jax
import jax.numpy as jnp
from jax import lax
from jax.experimental import pallas as pl
from jax.experimental.pallas import tpu as pltpu

SCALE = 128.0 ** -0.5


def _partial_body(q_ref, k_ref, v_ref, onum_ref, l_ref):
    n_heads = q_ref.shape[1]
    for h in range(n_heads):
        q = q_ref[0, h]
        k = k_ref[0, h]
        v = v_ref[0, h]
        s = lax.dot_general(
            q, k, (((1,), (1,)), ((), ())), preferred_element_type=jnp.float32
        )
        p = jnp.exp(s * SCALE)
        onum = lax.dot_general(
            p.astype(jnp.bfloat16), v, (((1,), (0,)), ((), ())),
            preferred_element_type=jnp.float32,
        )
        onum_ref[0, h] = onum
        l_ref[0, h] = jnp.sum(p, axis=1, keepdims=True)


def _combine_body(
    onum_ref, l_ref, osum_ref, lsum_ref, sbuf, obuf, lbuf, send_sems, recv_sems
):
    my_x = lax.axis_index("x")
    my_y = lax.axis_index("y")
    my_z = lax.axis_index("z")
    partners = (
        (1 - my_x, my_y, my_z),
        (my_x, 1 - my_y, my_z),
        (my_x, my_y, my_z + 1 - 2 * lax.rem(my_z, 2)),
        (my_x, my_y, lax.rem(my_z + 2, 4)),
    )

    barrier = pltpu.get_barrier_semaphore()
    for tgt in partners:
        pl.semaphore_signal(
            barrier, inc=1, device_id=tgt, device_id_type=pl.DeviceIdType.MESH
        )
    pl.semaphore_wait(barrier, len(partners))

    osum_ref[...] = onum_ref[...]
    lsum_ref[...] = l_ref[...]

    for step, tgt in enumerate(partners):
        sbuf[...] = osum_ref[...].astype(jnp.bfloat16)
        o_rdma = pltpu.make_async_remote_copy(
            src_ref=sbuf, dst_ref=obuf.at[step],
            send_sem=send_sems.at[2 * step], recv_sem=recv_sems.at[2 * step],
            device_id=tgt, device_id_type=pl.DeviceIdType.MESH,
        )
        l_rdma = pltpu.make_async_remote_copy(
            src_ref=lsum_ref, dst_ref=lbuf.at[step],
            send_sem=send_sems.at[2 * step + 1],
            recv_sem=recv_sems.at[2 * step + 1],
            device_id=tgt, device_id_type=pl.DeviceIdType.MESH,
        )
        o_rdma.start()
        l_rdma.start()
        o_rdma.wait()
        l_rdma.wait()
        osum_ref[...] = osum_ref[...] + obuf[step].astype(jnp.float32)
        lsum_ref[...] = lsum_ref[...] + lbuf[step]


def kernel(Q, K, V):
    B, SQ, H, D = Q.shape
    KV = K.shape[1]
    KVQ = KV // 4

    rank = 2 * lax.axis_index("x") + lax.axis_index("y")
    Kq = lax.dynamic_slice(K, (0, rank * KVQ, 0, 0), (B, KVQ, H, D))
    Vq = lax.dynamic_slice(V, (0, rank * KVQ, 0, 0), (B, KVQ, H, D))
    Kt = jnp.transpose(Kq, (0, 2, 1, 3)).astype(jnp.bfloat16)
    Vt = jnp.transpose(Vq, (0, 2, 1, 3)).astype(jnp.bfloat16)
    Qt = jnp.transpose(Q, (0, 2, 1, 3)).astype(jnp.bfloat16)

    onum, l = pl.pallas_call(
        _partial_body,
        grid=(B,),
        in_specs=[
            pl.BlockSpec((1, H, SQ, D), lambda bi: (bi, 0, 0, 0)),
            pl.BlockSpec((1, H, KVQ, D), lambda bi: (bi, 0, 0, 0)),
            pl.BlockSpec((1, H, KVQ, D), lambda bi: (bi, 0, 0, 0)),
        ],
        out_specs=[
            pl.BlockSpec((1, H, SQ, D), lambda bi: (bi, 0, 0, 0)),
            pl.BlockSpec((1, H, SQ, 1), lambda bi: (bi, 0, 0, 0)),
        ],
        out_shape=[
            jax.ShapeDtypeStruct((B, H, SQ, D), jnp.float32),
            jax.ShapeDtypeStruct((B, H, SQ, 1), jnp.float32),
        ],
    )(Qt, Kt, Vt)

    l_tile = l.reshape(B, H * SQ)

    osum, lsum = pl.pallas_call(
        _combine_body,
        in_specs=[
            pl.BlockSpec(memory_space=pltpu.VMEM),
            pl.BlockSpec(memory_space=pltpu.VMEM),
        ],
        out_specs=[
            pl.BlockSpec(memory_space=pltpu.VMEM),
            pl.BlockSpec(memory_space=pltpu.VMEM),
        ],
        out_shape=[
            jax.ShapeDtypeStruct((B, H, SQ, D), jnp.float32),
            jax.ShapeDtypeStruct((B, H * SQ), jnp.float32),
        ],
        scratch_shapes=[
            pltpu.VMEM((B, H, SQ, D), jnp.bfloat16),
            pltpu.VMEM((4, B, H, SQ, D), jnp.bfloat16),
            pltpu.VMEM((4, B, H * SQ), jnp.float32),
            pltpu.SemaphoreType.DMA((8,)),
            pltpu.SemaphoreType.DMA((8,)),
        ],
        compiler_params=pltpu.CompilerParams(collective_id=0),
    )(onum, l_tile)

    out = osum / lsum.reshape(B, H, SQ, 1)
    return jnp.transpose(out, (0, 2, 1, 3))
